# baseline (speedup 1.0000x reference)
import jax
import jax.numpy as jnp
from jax import lax
from jax.experimental import pallas as pl
from jax.experimental.pallas import tpu as pltpu
from jax.experimental.pallas import tpu_sc as plsc

B = 16384
L = 200
NC = 2
NS = 16
NW = NC * NS
B_SC = 8192
COLS_PER_W = B_SC // NW
NGROUP = COLS_PER_W // 16
LCHUNK = 8
NCHUNK = L // LCHUNK
PIPE = 6
BLK = 2048


def _sc_body(x_hbm, w_hbm, out_hbm, bb, wv, accv, sem):
    wid = lax.axis_index("s") * NC + lax.axis_index("c")
    base = wid * COLS_PER_W
    pltpu.sync_copy(w_hbm, wv)

    wvec = wv[pl.ds(0, 16)]
    ws = [jnp.broadcast_to(wvec[k], (16,)) for k in range(5)]

    def lookup(v):
        val = jnp.where(v == 1, ws[1], ws[0])
        val = jnp.where(v == 2, ws[2], val)
        val = jnp.where(v == 3, ws[3], val)
        return jnp.where(v == 4, ws[4], val)

    def issue(i):
        pltpu.async_copy(
            x_hbm.at[pl.ds(i * LCHUNK, LCHUNK), pl.ds(base, COLS_PER_W)],
            bb.at[pl.ds(i * LCHUNK, LCHUNK), :], sem)

    def zero_group(g, _):
        accv[pl.ds(g * 16, 16)] = jnp.zeros((16,), jnp.float32)
        return 0

    lax.fori_loop(0, NGROUP, zero_group, 0)

    for i in range(PIPE):
        issue(i)

    def chunk_body(i, _):
        pltpu.make_async_copy(
            x_hbm.at[pl.ds(0, LCHUNK), pl.ds(base, COLS_PER_W)],
            bb.at[pl.ds(0, LCHUNK), :], sem).wait()

        @pl.when(i + PIPE < NCHUNK)
        def _():
            issue(i + PIPE)

        def group_body(g, _):
            gl = g * 16
            vals = []
            for l in range(LCHUNK):
                v = bb[i * LCHUNK + l, pl.ds(gl, 16)]
                vals.append(lookup(v))
            while len(vals) > 1:
                vals = [a + b for a, b in zip(vals[::2], vals[1::2])]
            plsc.addupdate(accv.at[pl.ds(gl, 16)], vals[0])
            return 0

        lax.fori_loop(0, NGROUP, group_body, 0)
        return 0

    lax.fori_loop(0, NCHUNK, chunk_body, 0)
    pltpu.sync_copy(accv, out_hbm.at[pl.ds(base, COLS_PER_W)])


def _tc_body(w_ref, x_ref, o_ref):
    v = x_ref[...]
    w = [w_ref[0, k] for k in range(5)]
    val = jnp.where(v == 1, w[1], w[0])
    val = jnp.where(v == 2, w[2], val)
    val = jnp.where(v == 3, w[3], val)
    val = jnp.where(v == 4, w[4], val)
    o_ref[...] = jnp.sum(val, axis=0, keepdims=True)


@jax.jit
def _call(x_t, w128):
    mesh = plsc.VectorSubcoreMesh(core_axis_name="c", subcore_axis_name="s")
    sc = pl.kernel(
        _sc_body,
        out_type=jax.ShapeDtypeStruct((B_SC,), jnp.float32),
        mesh=mesh,
        scratch_types=[
            pltpu.VMEM((L, COLS_PER_W), jnp.int32),
            pltpu.VMEM((128,), jnp.float32),
            pltpu.VMEM((COLS_PER_W,), jnp.float32),
            pltpu.SemaphoreType.DMA,
        ],
        compiler_params=pltpu.CompilerParams(
            use_tc_tiling_on_sc=True, needs_layout_passes=False),
    )
    out_sc = sc(x_t, w128)

    ntc = (B - B_SC) // BLK
    out_tc = pl.pallas_call(
        _tc_body,
        grid=(ntc,),
        in_specs=[
            pl.BlockSpec((1, 128), lambda j: (0, 0)),
            pl.BlockSpec((L, BLK), lambda j: (0, B_SC // BLK + j)),
        ],
        out_specs=pl.BlockSpec((1, BLK), lambda j: (0, j)),
        out_shape=jax.ShapeDtypeStruct((1, B - B_SC), jnp.float32),
    )(w128.reshape(1, 128), x_t)

    return jnp.concatenate([out_sc, out_tc.reshape(B - B_SC)])


def kernel(x, emb_weight):
    w128 = jnp.zeros((128,), jnp.float32).at[:5].set(emb_weight[:, 0])
    out = _call(x.T, w128)
    return out.reshape(B, 1)


# speedup vs baseline: 1.0099x; 1.0099x over previous
import jax
import jax.numpy as jnp
from jax import lax
from jax.experimental import pallas as pl
from jax.experimental.pallas import tpu as pltpu
from jax.experimental.pallas import tpu_sc as plsc

B = 16384
L = 200
NC = 2
NS = 16
NW = NC * NS
B_SC = 8192
COLS_PER_W = B_SC // NW
NGROUP = COLS_PER_W // 16
LCHUNK = 8
NCHUNK = L // LCHUNK
PIPE = 6
BLK = 2048


def _sc_body(x_hbm, w_hbm, out_hbm, bb, wv, accv, sem):
    wid = lax.axis_index("s") * NC + lax.axis_index("c")
    base = wid * COLS_PER_W
    pltpu.sync_copy(w_hbm, wv)

    wvec = wv[pl.ds(0, 16)]
    ws = [jnp.broadcast_to(wvec[k], (16,)) for k in range(5)]

    def lookup(v):
        val = jnp.where(v == 1, ws[1], ws[0])
        val = jnp.where(v == 2, ws[2], val)
        val = jnp.where(v == 3, ws[3], val)
        return jnp.where(v == 4, ws[4], val)

    def issue(i):
        pltpu.async_copy(
            x_hbm.at[pl.ds(i * LCHUNK, LCHUNK), pl.ds(base, COLS_PER_W)],
            bb.at[pl.ds(i * LCHUNK, LCHUNK), :], sem)

    def zero_group(g, _):
        accv[pl.ds(g * 16, 16)] = jnp.zeros((16,), jnp.float32)
        return 0

    lax.fori_loop(0, NGROUP, zero_group, 0)

    for i in range(PIPE):
        issue(i)

    def chunk_body(i, _):
        pltpu.make_async_copy(
            x_hbm.at[pl.ds(0, LCHUNK), pl.ds(base, COLS_PER_W)],
            bb.at[pl.ds(0, LCHUNK), :], sem).wait()

        @pl.when(i + PIPE < NCHUNK)
        def _():
            issue(i + PIPE)

        def group_body(g, _):
            gl = g * 16
            vals = []
            for l in range(LCHUNK):
                v = bb[i * LCHUNK + l, pl.ds(gl, 16)]
                vals.append(lookup(v))
            while len(vals) > 1:
                vals = [a + b for a, b in zip(vals[::2], vals[1::2])]
            plsc.addupdate(accv.at[pl.ds(gl, 16)], vals[0])
            return 0

        lax.fori_loop(0, NGROUP, group_body, 0)
        return 0

    lax.fori_loop(0, NCHUNK, chunk_body, 0)
    pltpu.sync_copy(accv, out_hbm.at[pl.ds(base, COLS_PER_W)])


def _tc_body(c_ref, x_ref, o_ref):
    vf = x_ref[...].astype(jnp.float32)
    c = [c_ref[0, k] for k in range(5)]
    val = c[4] * vf + c[3]
    val = val * vf + c[2]
    val = val * vf + c[1]
    val = val * vf + c[0]
    o_ref[...] = jnp.sum(val, axis=0, keepdims=True)


@jax.jit
def _call(x_t, w128, c128):
    mesh = plsc.VectorSubcoreMesh(core_axis_name="c", subcore_axis_name="s")
    sc = pl.kernel(
        _sc_body,
        out_type=jax.ShapeDtypeStruct((B_SC,), jnp.float32),
        mesh=mesh,
        scratch_types=[
            pltpu.VMEM((L, COLS_PER_W), jnp.int32),
            pltpu.VMEM((128,), jnp.float32),
            pltpu.VMEM((COLS_PER_W,), jnp.float32),
            pltpu.SemaphoreType.DMA,
        ],
        compiler_params=pltpu.CompilerParams(
            use_tc_tiling_on_sc=True, needs_layout_passes=False),
    )
    out_sc = sc(x_t, w128)

    ntc = (B - B_SC) // BLK
    out_tc = pl.pallas_call(
        _tc_body,
        grid=(ntc,),
        in_specs=[
            pl.BlockSpec((1, 128), lambda j: (0, 0)),
            pl.BlockSpec((L, BLK), lambda j: (0, B_SC // BLK + j)),
        ],
        out_specs=pl.BlockSpec((1, BLK), lambda j: (0, j)),
        out_shape=jax.ShapeDtypeStruct((1, B - B_SC), jnp.float32),
    )(c128.reshape(1, 128), x_t)

    return jnp.concatenate([out_sc, out_tc.reshape(B - B_SC)])


_VINV = jnp.array([
    [24, 0, 0, 0, 0],
    [-50, 96, -72, 32, -6],
    [35, -104, 114, -56, 11],
    [-10, 36, -48, 28, -6],
    [1, -4, 6, -4, 1],
], dtype=jnp.float32) / 24.0


def kernel(x, emb_weight):
    coef = _VINV @ emb_weight[:, 0]
    w128 = jnp.zeros((128,), jnp.float32).at[:5].set(emb_weight[:, 0])
    c128 = jnp.zeros((128,), jnp.float32).at[:5].set(coef)
    out = _call(x.T, w128, c128)
    return out.reshape(B, 1)
